# baseline (speedup 1.0000x reference)
import functools

import jax
import jax.numpy as jnp
from jax import lax
from jax.experimental import pallas as pl
from jax.experimental.pallas import tpu as pltpu
from jax.experimental.pallas import tpu_sc as plsc

N = 100000
D = 256
LANES = 16
NCHUNK = D // LANES

BLK = 40
NPAR = 32
NCH = 88
TREE_BLKS = 1250
NBLK = 2500
NWORKERS = 32
NBUF = 3

TC_R = 2000


def _sc_agg_body(h_hbm, out_hbm, par_v, ch_v, out_v, sem_par, sem_ch, sem_out):
    wid = lax.axis_index("s") * 2 + lax.axis_index("c")

    def tree_bi(t):
        return t * NWORKERS + wid

    def leaf_bi(t):
        return TREE_BLKS + t * NWORKERS + wid

    def par_base(r0):
        return jnp.maximum((((r0 >> 1) - 1) >> 3) << 3, 0)

    def in_copies(bi, b, with_ch):
        r0 = bi * BLK
        p0 = pl.multiple_of(par_base(r0), 8)
        cps = [pltpu.make_async_copy(h_hbm.at[pl.ds(p0, NPAR)],
                                     par_v.at[b, pl.ds(0, NPAR)], sem_par)]
        if with_ch:
            c0 = pl.multiple_of(jnp.minimum(2 * r0, N - NCH), 8)
            cps.append(pltpu.make_async_copy(
                h_hbm.at[pl.ds(c0, NCH)], ch_v.at[b, pl.ds(0, NCH)], sem_ch))
        return cps

    def out_copy(bi, b):
        return pltpu.make_async_copy(
            out_v.at[b],
            out_hbm.at[pl.ds(pl.multiple_of(bi * BLK, 8), BLK)], sem_out)

    def pipeline(bi_of, n, with_ch, compute):
        def buf(t):
            return t - (t // NBUF) * NBUF

        def fire(t):
            for cp in in_copies(bi_of(t), buf(t), with_ch):
                cp.start()

        def wait_in(t):
            for cp in in_copies(bi_of(t), buf(t), with_ch):
                cp.wait()

        fire(0)
        fire(1)

        def body(t, _):
            wait_in(t)

            @pl.when(t + 2 < n)
            def _():
                fire(t + 2)

            @pl.when(t >= NBUF)
            def _():
                out_copy(bi_of(t - NBUF), buf(t)).wait()

            compute(bi_of(t), buf(t))
            out_copy(bi_of(t), buf(t)).start()
            return 0

        lax.fori_loop(0, n, body, 0)
        out_copy(bi_of(n - 3), buf(n - 3)).wait()
        out_copy(bi_of(n - 2), buf(n - 2)).wait()
        out_copy(bi_of(n - 1), buf(n - 1)).wait()

    def tree_compute(bi, b):
        r0 = bi * BLK
        p0 = par_base(r0)
        ch_off = 2 * r0 - jnp.minimum(2 * r0, N - NCH)

        def node(k, _):
            g = r0 + k
            pk = jnp.maximum(((g - 1) >> 1) - p0, 0)
            i1 = jnp.minimum(2 * k + 1 + ch_off, NCH - 1)
            i2 = jnp.minimum(2 * k + 2 + ch_off, NCH - 1)
            mp = jnp.where(g > 0, 1.0, 0.0).astype(jnp.float32)
            m1 = jnp.where(2 * g + 1 < N, 1.0, 0.0).astype(jnp.float32)
            m2 = jnp.where(2 * g + 2 < N, 1.0, 0.0).astype(jnp.float32)
            for c in range(NCHUNK):
                sl = pl.ds(c * LANES, LANES)
                out_v[b, k, sl] = (mp * par_v[b, pk, sl]
                                   + m1 * ch_v[b, i1, sl]
                                   + m2 * ch_v[b, i2, sl])
            return 0

        lax.fori_loop(0, BLK, node, 0)

    def leaf_compute(bi, b):
        r0 = bi * BLK
        p0 = par_base(r0)

        def node(k, _):
            pk = ((r0 + k - 1) >> 1) - p0
            for c in range(NCHUNK):
                sl = pl.ds(c * LANES, LANES)
                out_v[b, k, sl] = par_v[b, pk, sl]
            return 0

        lax.fori_loop(0, BLK, node, 0)

    n_tree = (TREE_BLKS // NWORKERS) + jnp.where(
        wid < TREE_BLKS % NWORKERS, 1, 0)
    n_leaf = ((NBLK - TREE_BLKS) // NWORKERS) + jnp.where(
        wid < (NBLK - TREE_BLKS) % NWORKERS, 1, 0)
    pipeline(tree_bi, n_tree, True, tree_compute)
    pipeline(leaf_bi, n_leaf, False, leaf_compute)


@jax.jit
def _sc_agg(h):
    mesh = plsc.VectorSubcoreMesh(core_axis_name="c", subcore_axis_name="s")
    return pl.kernel(
        _sc_agg_body,
        out_type=jax.ShapeDtypeStruct((N, D), jnp.float32),
        mesh=mesh,
        scratch_types=[
            pltpu.VMEM((NBUF, NPAR, D), jnp.float32),
            pltpu.VMEM((NBUF, NCH, D), jnp.float32),
            pltpu.VMEM((NBUF, BLK, D), jnp.float32),
            pltpu.SemaphoreType.DMA,
            pltpu.SemaphoreType.DMA,
            pltpu.SemaphoreType.DMA,
        ],
    )(h)


def _tc_dense_body(act, msg_ref, h_ref, w_ref, b_ref, o_ref):
    gi = pl.program_id(0) * TC_R + lax.broadcasted_iota(jnp.int32, (TC_R, 1), 0)
    third = jnp.float32(1.0 / 3.0)
    inv = jnp.where(
        gi >= 50000, jnp.float32(0.5),
        jnp.where((gi == 0) | (gi == 49999), third, jnp.float32(0.25)))
    u = (msg_ref[...] + h_ref[...]) * inv
    y = jnp.dot(u, w_ref[...], preferred_element_type=jnp.float32) + b_ref[...]
    o_ref[...] = jnp.maximum(y, 0.0) if act else y


@functools.partial(jax.jit, static_argnames=("act",))
def _tc_dense(msg, h, w, b, act):
    grid = N // TC_R
    return pl.pallas_call(
        functools.partial(_tc_dense_body, act),
        grid=(grid,),
        in_specs=[
            pl.BlockSpec((TC_R, D), lambda i: (i, 0)),
            pl.BlockSpec((TC_R, D), lambda i: (i, 0)),
            pl.BlockSpec((D, D), lambda i: (0, 0)),
            pl.BlockSpec((1, D), lambda i: (0, 0)),
        ],
        out_specs=pl.BlockSpec((TC_R, D), lambda i: (i, 0)),
        out_shape=jax.ShapeDtypeStruct((N, D), jnp.float32),
    )(msg, h, w, b)


def kernel(x, W1, b1, W2, b2):
    b1r = b1.reshape(1, D)
    b2r = b2.reshape(1, D)
    msg1 = _sc_agg(x)
    h1 = _tc_dense(msg1, x, W1, b1r, act=True)
    msg2 = _sc_agg(h1)
    return _tc_dense(msg2, h1, W2, b2r, act=False)


# speedup vs baseline: 1.7585x; 1.7585x over previous
import functools

import jax
import jax.numpy as jnp
from jax import lax
from jax.experimental import pallas as pl
from jax.experimental.pallas import tpu as pltpu
from jax.experimental.pallas import tpu_sc as plsc

N = 100000
D = 256
LANES = 16
NCHUNK = D // LANES

BLK = 40
NPAR = 32
NCH = 88
TREE_BLKS = 1250
NBLK = 2500
NWORKERS = 32
NBUF = 3

TC_R = 2000


def _sc_agg_body(h_hbm, out_hbm, par_v, ch_v, out_v, sem_par, sem_ch, sem_out):
    wid = lax.axis_index("s") * 2 + lax.axis_index("c")

    def tree_bi(t):
        return t * NWORKERS + wid

    def leaf_bi(t):
        return TREE_BLKS + t * NWORKERS + wid

    def par_base(r0):
        return jnp.maximum((((r0 >> 1) - 1) >> 3) << 3, 0)

    def in_copies(bi, b, with_ch):
        r0 = bi * BLK
        p0 = pl.multiple_of(par_base(r0), 8)
        cps = [pltpu.make_async_copy(h_hbm.at[pl.ds(p0, NPAR)],
                                     par_v.at[b, pl.ds(0, NPAR)], sem_par)]
        if with_ch:
            c0 = pl.multiple_of(jnp.minimum(2 * r0, N - NCH), 8)
            cps.append(pltpu.make_async_copy(
                h_hbm.at[pl.ds(c0, NCH)], ch_v.at[b, pl.ds(0, NCH)], sem_ch))
        return cps

    def out_copy(bi, b):
        return pltpu.make_async_copy(
            out_v.at[b],
            out_hbm.at[pl.ds(pl.multiple_of(bi * BLK, 8), BLK)], sem_out)

    def pipeline(bi_of, n, with_ch, compute):
        def buf(t):
            return t - (t // NBUF) * NBUF

        def fire(t):
            for cp in in_copies(bi_of(t), buf(t), with_ch):
                cp.start()

        def wait_in(t):
            for cp in in_copies(bi_of(t), buf(t), with_ch):
                cp.wait()

        fire(0)
        fire(1)

        def body(t, _):
            wait_in(t)

            @pl.when(t + 2 < n)
            def _():
                fire(t + 2)

            @pl.when(t >= NBUF)
            def _():
                out_copy(bi_of(t - NBUF), buf(t)).wait()

            compute(bi_of(t), buf(t))
            out_copy(bi_of(t), buf(t)).start()
            return 0

        lax.fori_loop(0, n, body, 0)
        out_copy(bi_of(n - 3), buf(n - 3)).wait()
        out_copy(bi_of(n - 2), buf(n - 2)).wait()
        out_copy(bi_of(n - 1), buf(n - 1)).wait()

    GRP = 4

    def tree_compute(bi, b):
        r0 = bi * BLK
        p0 = par_base(r0)
        ch_off = 2 * r0 - jnp.minimum(2 * r0, N - NCH)
        edge = (bi == 0) | (bi == TREE_BLKS - 1)

        def edge_node(k, _):
            g = r0 + k
            pk = jnp.maximum(((g - 1) >> 1) - p0, 0)
            i1 = jnp.minimum(2 * k + 1 + ch_off, NCH - 1)
            i2 = jnp.minimum(2 * k + 2 + ch_off, NCH - 1)
            mp = jnp.where(g > 0, 1.0, 0.0).astype(jnp.float32)
            m1 = jnp.where(2 * g + 1 < N, 1.0, 0.0).astype(jnp.float32)
            m2 = jnp.where(2 * g + 2 < N, 1.0, 0.0).astype(jnp.float32)
            for c in range(NCHUNK):
                sl = pl.ds(c * LANES, LANES)
                out_v[b, k, sl] = (mp * par_v[b, pk, sl]
                                   + m1 * ch_v[b, i1, sl]
                                   + m2 * ch_v[b, i2, sl])
            return 0

        def clean_node(k, _):
            g = r0 + k
            pk = ((g - 1) >> 1) - p0
            for c0 in range(0, NCHUNK, GRP):
                sls = [pl.ds((c0 + j) * LANES, LANES) for j in range(GRP)]
                ps = [par_v[b, pk, sl] for sl in sls]
                a1 = [ch_v[b, 2 * k + 1, sl] for sl in sls]
                a2 = [ch_v[b, 2 * k + 2, sl] for sl in sls]
                for j, sl in enumerate(sls):
                    out_v[b, k, sl] = ps[j] + (a1[j] + a2[j])
            return 0

        @pl.when(edge)
        def _():
            lax.fori_loop(0, BLK, edge_node, 0)

        @pl.when(jnp.logical_not(edge))
        def _():
            lax.fori_loop(0, BLK, clean_node, 0)

    def leaf_compute(bi, b):
        r0 = bi * BLK
        p0 = par_base(r0)

        def node(k, _):
            pk = ((r0 + k - 1) >> 1) - p0
            for c0 in range(0, NCHUNK, GRP):
                sls = [pl.ds((c0 + j) * LANES, LANES) for j in range(GRP)]
                ps = [par_v[b, pk, sl] for sl in sls]
                for j, sl in enumerate(sls):
                    out_v[b, k, sl] = ps[j]
            return 0

        lax.fori_loop(0, BLK, node, 0)

    n_tree = (TREE_BLKS // NWORKERS) + jnp.where(
        wid < TREE_BLKS % NWORKERS, 1, 0)
    n_leaf = ((NBLK - TREE_BLKS) // NWORKERS) + jnp.where(
        wid < (NBLK - TREE_BLKS) % NWORKERS, 1, 0)
    pipeline(tree_bi, n_tree, True, tree_compute)
    pipeline(leaf_bi, n_leaf, False, leaf_compute)


@jax.jit
def _sc_agg(h):
    mesh = plsc.VectorSubcoreMesh(core_axis_name="c", subcore_axis_name="s")
    return pl.kernel(
        _sc_agg_body,
        out_type=jax.ShapeDtypeStruct((N, D), jnp.float32),
        mesh=mesh,
        scratch_types=[
            pltpu.VMEM((NBUF, NPAR, D), jnp.float32),
            pltpu.VMEM((NBUF, NCH, D), jnp.float32),
            pltpu.VMEM((NBUF, BLK, D), jnp.float32),
            pltpu.SemaphoreType.DMA,
            pltpu.SemaphoreType.DMA,
            pltpu.SemaphoreType.DMA,
        ],
    )(h)


def _tc_dense_body(act, msg_ref, h_ref, w_ref, b_ref, o_ref):
    gi = pl.program_id(0) * TC_R + lax.broadcasted_iota(jnp.int32, (TC_R, 1), 0)
    third = jnp.float32(1.0 / 3.0)
    inv = jnp.where(
        gi >= 50000, jnp.float32(0.5),
        jnp.where((gi == 0) | (gi == 49999), third, jnp.float32(0.25)))
    u = (msg_ref[...] + h_ref[...]) * inv
    y = jnp.dot(u, w_ref[...], preferred_element_type=jnp.float32) + b_ref[...]
    o_ref[...] = jnp.maximum(y, 0.0) if act else y


@functools.partial(jax.jit, static_argnames=("act",))
def _tc_dense(msg, h, w, b, act):
    grid = N // TC_R
    return pl.pallas_call(
        functools.partial(_tc_dense_body, act),
        grid=(grid,),
        in_specs=[
            pl.BlockSpec((TC_R, D), lambda i: (i, 0)),
            pl.BlockSpec((TC_R, D), lambda i: (i, 0)),
            pl.BlockSpec((D, D), lambda i: (0, 0)),
            pl.BlockSpec((1, D), lambda i: (0, 0)),
        ],
        out_specs=pl.BlockSpec((TC_R, D), lambda i: (i, 0)),
        out_shape=jax.ShapeDtypeStruct((N, D), jnp.float32),
    )(msg, h, w, b)


def kernel(x, W1, b1, W2, b2):
    b1r = b1.reshape(1, D)
    b2r = b2.reshape(1, D)
    msg1 = _sc_agg(x)
    h1 = _tc_dense(msg1, x, W1, b1r, act=True)
    msg2 = _sc_agg(h1)
    return _tc_dense(msg2, h1, W2, b2r, act=False)
